# baseline (speedup 1.0000x reference)
import numpy as np

import jax
import jax.numpy as jnp
from jax.experimental import pallas as pl
from jax.experimental.pallas import tpu as pltpu

_BF = jnp.bfloat16


def _conv_srcmap(si, so, stride, cout):
    hwo = so * so
    m = np.full((9, hwo), -1, np.int32)
    for kh in range(3):
        for kw in range(3):
            t = kh * 3 + kw
            for r in range(so):
                ir = stride * r + kh - 1
                if not 0 <= ir < si:
                    continue
                for c in range(so):
                    ic = stride * c + kw - 1
                    if 0 <= ic < si:
                        m[t, r * so + c] = ir * si + ic
    return np.tile(m, (1, cout))


def _conv912_srcmap(cout):
    m = np.full((9, 256), -1, np.int32)
    for kh in range(3):
        for kw in range(3):
            t = kh * 3 + kw
            for r in range(16):
                ir = r + kh - 1
                if not 0 <= ir < 16:
                    continue
                for c in range(16):
                    ic = c + kw - 1
                    if 0 <= ic < 16:
                        m[t, r * 16 + c] = (ir // 2) * 8 + (ic // 2)
    return np.tile(m, (1, cout))


def _up4_idmap():
    m = np.zeros(64, np.int32)
    for r in range(8):
        for c in range(8):
            m[r * 8 + c] = (r // 2) * 4 + (c // 2)
    return np.tile(m, 16)[None, :]


_SRC0 = _conv_srcmap(32, 16, 2, 8)
_SRC1 = _conv_srcmap(16, 8, 2, 16)
_SRC2 = _conv_srcmap(8, 4, 2, 32)
_SRC710 = _conv_srcmap(8, 8, 1, 16)
_SRC912 = _conv912_srcmap(4)
_ID5A = _up4_idmap()
_ID5B = (np.arange(1024, dtype=np.int32) % 64)[None, :]
_IDD0 = ((np.arange(2880, dtype=np.int32) // 15) % 64)[None, :]
_IDD1 = ((np.arange(720, dtype=np.int32) // 15) % 16)[None, :]

_GEOM_CONV = ((1024, 3, 2048), (256, 8, 1024), (64, 16, 512),
              (64, 16, 1024), (64, 16, 1024))
_GEOM_ID = ((16, 32, 1024), (64, 16, 1024),
            (64, 16, 2880), (16, 32, 720))


def _wrep_conv(w, hwo):
    t, co, ci = w.shape
    r = jnp.repeat(w.transpose(0, 2, 1), hwo, axis=2)
    return r.reshape(t * ci, co * hwo)


def _wrep_det(wd, side):
    ci = wd.shape[1]
    a = wd.reshape(3, 15, ci).transpose(2, 0, 1)[:, :, None, :]
    return jnp.broadcast_to(a, (ci, 3, side * side, 15)).reshape(
        ci, 3 * side * side * 15)


def _det_bias_row(bd, side):
    hw = side * side
    return jnp.broadcast_to(bd.reshape(3, 1, 1, 15),
                            (3, side, side, 15)).reshape(1, 3 * hw * 15)


def _silu_bf(v):
    return (v * pl.reciprocal(1.0 + jnp.exp(-v), approx=True)).astype(_BF)


def _build_conv(scratch_ref, wrep_ref, src_ref, hwi, cin, cols):
    ch = min(256, hwi)

    def chunk(i, off, rows):
        iloc = off + jax.lax.broadcasted_iota(jnp.int32, (rows, cols), 0)
        acc = jnp.zeros((rows, cols), jnp.float32)
        for t in range(9):
            srow = src_ref[t:t + 1, :]
            r = t * cin + i
            wrow = wrep_ref[r:r + 1, :]
            acc = acc + jnp.where(iloc == srow,
                                  jnp.broadcast_to(wrow, (rows, cols)),
                                  jnp.zeros((rows, cols), jnp.float32))
        return acc.astype(_BF)

    for i in range(cin):
        if hwi > ch:
            def body(k, _, _i=i):
                off = pl.multiple_of(k * ch, ch)
                scratch_ref[pl.ds(_i * hwi + off, ch), :] = chunk(_i, off, ch)
                return 0
            jax.lax.fori_loop(0, hwi // ch, body, 0)
        else:
            scratch_ref[i * hwi:(i + 1) * hwi, :] = chunk(i, 0, hwi)


def _build_id(scratch_ref, wrep_ref, id_ref, hwi, cin, cols):
    iloc = jax.lax.broadcasted_iota(jnp.int32, (hwi, cols), 0)
    sel = iloc == id_ref[...]
    for i in range(cin):
        wrow = wrep_ref[i:i + 1, :]
        scratch_ref[i * hwi:(i + 1) * hwi, :] = jnp.where(
            sel, jnp.broadcast_to(wrow, (hwi, cols)),
            jnp.zeros((hwi, cols), jnp.float32)).astype(_BF)


def _mcnet_kernel(x0_ref,
                  wr0_ref, wr1_ref, wr2_ref, wr710_ref, wr912_ref,
                  wr5a_ref, wr5b_ref, wrd0_ref, wrd1_ref,
                  s0_ref, s1_ref, s2_ref, s710_ref, s912_ref,
                  i5a_ref, i5b_ref, id0_ref, id1_ref,
                  b0_ref, b1_ref, b2_ref, b5_ref,
                  bd0_ref, bd1_ref, b710_ref, b912_ref,
                  det0_ref, det1_ref, da_ref, ll_ref,
                  W0, W1, W2, W710, W912, W5a, W5b, Wd0, Wd1):
    f32 = jnp.float32
    pid = pl.program_id(0)

    @pl.when(pid == 0)
    def _build():
        convs = ((W0, wr0_ref, s0_ref), (W1, wr1_ref, s1_ref),
                 (W2, wr2_ref, s2_ref), (W710, wr710_ref, s710_ref),
                 (W912, wr912_ref, s912_ref))
        for (sc, wr, sr), g in zip(convs, _GEOM_CONV):
            _build_conv(sc, wr, sr, *g)
        ids = ((W5a, wr5a_ref, i5a_ref), (W5b, wr5b_ref, i5b_ref),
               (Wd0, wrd0_ref, id0_ref), (Wd1, wrd1_ref, id1_ref))
        for (sc, wr, ir), g in zip(ids, _GEOM_ID):
            _build_id(sc, wr, ir, *g)

    @pl.when(pid > 0)
    def _compute():
        def dot(a, b_ref):
            return jnp.dot(a, b_ref[...], preferred_element_type=f32)

        a0 = _silu_bf(dot(x0_ref[...].astype(_BF), W0) + b0_ref[...])
        a1 = _silu_bf(dot(a0, W1) + b1_ref[...])
        a2 = _silu_bf(dot(a1, W2) + b2_ref[...])
        a5 = _silu_bf(dot(a2, W5a) + dot(a1, W5b) + b5_ref[...])
        det0_ref[...] = dot(a5, Wd0) + bd0_ref[...]
        det1_ref[...] = dot(a2, Wd1) + bd1_ref[...]
        a710 = _silu_bf(dot(a5, W710) + b710_ref[...])
        seg = 1.0 / (1.0 + jnp.exp(-(dot(a710, W912) + b912_ref[...])))
        da_ref[...] = seg[:, 0:512]
        ll_ref[...] = seg[:, 512:1024]


def _const_spec(shape):
    return pl.BlockSpec(shape, lambda b: (0,) * len(shape))


def kernel(x, w0, b0, w1, b1, w2, b2, w5, b5, wd0, bd0, wd1, bd1,
           w710, b710, w912, b912):
    f32 = jnp.float32
    x = x.astype(f32)
    n = x.shape[0]
    bb = 256 if n % 256 == 0 else n
    nblk = n // bb

    w0r = w0.reshape(8, 9, 3).transpose(1, 0, 2)
    wreps = (_wrep_conv(w0r, 256), _wrep_conv(w1, 64), _wrep_conv(w2, 16),
             _wrep_conv(w710, 64), _wrep_conv(w912, 256),
             jnp.repeat(w5[:, :32].T, 64, axis=1),
             jnp.repeat(w5[:, 32:48].T, 64, axis=1),
             _wrep_det(wd0, 8), _wrep_det(wd1, 4))
    srcs = tuple(jnp.asarray(a) for a in
                 (_SRC0, _SRC1, _SRC2, _SRC710, _SRC912,
                  _ID5A, _ID5B, _IDD0, _IDD1))

    def brow(b, rep):
        return jnp.repeat(b.astype(f32), rep)[None, :]

    biases = (brow(b0, 256), brow(b1, 64), brow(b2, 16), brow(b5, 64),
              _det_bias_row(bd0, 8), _det_bias_row(bd1, 4),
              brow(b710, 64), brow(b912, 256))

    x0 = x.reshape(n, 3 * 1024)

    def xmap(b):
        return (jnp.maximum(b - 1, 0), 0)

    det0, det1, da, ll = pl.pallas_call(
        _mcnet_kernel,
        grid=(nblk + 1,),
        in_specs=([pl.BlockSpec((bb, 3072), xmap)]
                  + [_const_spec(a.shape) for a in wreps]
                  + [_const_spec(a.shape) for a in srcs]
                  + [_const_spec(a.shape) for a in biases]),
        out_specs=(
            pl.BlockSpec((bb, 2880), xmap),
            pl.BlockSpec((bb, 720), xmap),
            pl.BlockSpec((bb, 512), xmap),
            pl.BlockSpec((bb, 512), xmap),
        ),
        out_shape=(
            jax.ShapeDtypeStruct((n, 2880), f32),
            jax.ShapeDtypeStruct((n, 720), f32),
            jax.ShapeDtypeStruct((n, 512), f32),
            jax.ShapeDtypeStruct((n, 512), f32),
        ),
        scratch_shapes=[
            pltpu.VMEM((3072, 2048), _BF), pltpu.VMEM((2048, 1024), _BF),
            pltpu.VMEM((1024, 512), _BF), pltpu.VMEM((1024, 1024), _BF),
            pltpu.VMEM((1024, 1024), _BF), pltpu.VMEM((512, 1024), _BF),
            pltpu.VMEM((1024, 1024), _BF), pltpu.VMEM((1024, 2880), _BF),
            pltpu.VMEM((512, 720), _BF),
        ],
        compiler_params=pltpu.CompilerParams(
            dimension_semantics=("arbitrary",),
            vmem_limit_bytes=56 * 1024 * 1024),
    )(x0, *wreps, *srcs, *biases)

    det_out = [det0.reshape(n, 3, 8, 8, 15), det1.reshape(n, 3, 4, 4, 15)]
    return [det_out, da.reshape(n, 2, 16, 16), ll.reshape(n, 2, 16, 16)]


# speedup vs baseline: 1.0113x; 1.0113x over previous
import numpy as np

import jax
import jax.numpy as jnp
from jax.experimental import pallas as pl
from jax.experimental.pallas import tpu as pltpu

_BF = jnp.bfloat16


def _conv_srcmap(si, so, stride, cout):
    hwo = so * so
    m = np.full((9, hwo), -1, np.int32)
    for kh in range(3):
        for kw in range(3):
            t = kh * 3 + kw
            for r in range(so):
                ir = stride * r + kh - 1
                if not 0 <= ir < si:
                    continue
                for c in range(so):
                    ic = stride * c + kw - 1
                    if 0 <= ic < si:
                        m[t, r * so + c] = ir * si + ic
    return np.tile(m, (1, cout))


def _conv912_srcmap(cout):
    m = np.full((9, 256), -1, np.int32)
    for kh in range(3):
        for kw in range(3):
            t = kh * 3 + kw
            for r in range(16):
                ir = r + kh - 1
                if not 0 <= ir < 16:
                    continue
                for c in range(16):
                    ic = c + kw - 1
                    if 0 <= ic < 16:
                        m[t, r * 16 + c] = (ir // 2) * 8 + (ic // 2)
    return np.tile(m, (1, cout))


def _up4_idmap():
    m = np.zeros(64, np.int32)
    for r in range(8):
        for c in range(8):
            m[r * 8 + c] = (r // 2) * 4 + (c // 2)
    return np.tile(m, 16)[None, :]


_SRC0 = _conv_srcmap(32, 16, 2, 8)
_SRC1 = _conv_srcmap(16, 8, 2, 16)
_SRC2 = _conv_srcmap(8, 4, 2, 32)
_SRC710 = _conv_srcmap(8, 8, 1, 16)
_SRC912 = _conv912_srcmap(4)
_ID5A = _up4_idmap()
_ID5B = (np.arange(1024, dtype=np.int32) % 64)[None, :]
_IDD0 = ((np.arange(2880, dtype=np.int32) // 15) % 64)[None, :]
_IDD1 = ((np.arange(720, dtype=np.int32) // 15) % 16)[None, :]

_GEOM_CONV = ((1024, 3, 2048), (256, 8, 1024), (64, 16, 512),
              (64, 16, 1024), (64, 16, 1024))
_GEOM_ID = ((16, 32, 1024), (64, 16, 1024),
            (64, 16, 2880), (16, 32, 720))


def _wrep_conv(w, hwo):
    t, co, ci = w.shape
    r = jnp.repeat(w.transpose(0, 2, 1), hwo, axis=2)
    return r.reshape(t * ci, co * hwo)


def _wrep_det(wd, side):
    ci = wd.shape[1]
    a = wd.reshape(3, 15, ci).transpose(2, 0, 1)[:, :, None, :]
    return jnp.broadcast_to(a, (ci, 3, side * side, 15)).reshape(
        ci, 3 * side * side * 15)


def _det_bias_row(bd, side):
    hw = side * side
    return jnp.broadcast_to(bd.reshape(3, 1, 1, 15),
                            (3, side, side, 15)).reshape(1, 3 * hw * 15)


def _silu_bf(v):
    return (v * pl.reciprocal(1.0 + jnp.exp(-v), approx=True)).astype(_BF)


def _build_conv(scratch_ref, wrep_ref, src_ref, hwi, cin, cols):
    ch = min(128, hwi)

    def chunk(i, off, rows):
        iloc = off + jax.lax.broadcasted_iota(jnp.int32, (rows, cols), 0)
        acc = jnp.zeros((rows, cols), jnp.float32)
        for t in range(9):
            srow = src_ref[t:t + 1, :]
            r = t * cin + i
            wrow = wrep_ref[r:r + 1, :]
            acc = acc + jnp.where(iloc == srow,
                                  jnp.broadcast_to(wrow, (rows, cols)),
                                  jnp.zeros((rows, cols), jnp.float32))
        return acc.astype(_BF)

    for i in range(cin):
        if hwi > ch:
            def body(k, _, _i=i):
                off = pl.multiple_of(k * ch, ch)
                scratch_ref[pl.ds(_i * hwi + off, ch), :] = chunk(_i, off, ch)
                return 0
            jax.lax.fori_loop(0, hwi // ch, body, 0)
        else:
            scratch_ref[i * hwi:(i + 1) * hwi, :] = chunk(i, 0, hwi)


def _build_id(scratch_ref, wrep_ref, id_ref, hwi, cin, cols):
    iloc = jax.lax.broadcasted_iota(jnp.int32, (hwi, cols), 0)
    sel = iloc == id_ref[...]
    for i in range(cin):
        wrow = wrep_ref[i:i + 1, :]
        scratch_ref[i * hwi:(i + 1) * hwi, :] = jnp.where(
            sel, jnp.broadcast_to(wrow, (hwi, cols)),
            jnp.zeros((hwi, cols), jnp.float32)).astype(_BF)


def _mcnet_kernel(x0_ref,
                  wr0_ref, wr1_ref, wr2_ref, wr710_ref, wr912_ref,
                  wr5a_ref, wr5b_ref, wrd0_ref, wrd1_ref,
                  s0_ref, s1_ref, s2_ref, s710_ref, s912_ref,
                  i5a_ref, i5b_ref, id0_ref, id1_ref,
                  b0_ref, b1_ref, b2_ref, b5_ref,
                  bd0_ref, bd1_ref, b710_ref, b912_ref,
                  det0_ref, det1_ref, da_ref, ll_ref,
                  W0, W1, W2, W710, W912, W5a, W5b, Wd0, Wd1):
    f32 = jnp.float32
    pid = pl.program_id(0)

    @pl.when(pid == 0)
    def _build():
        convs = ((W0, wr0_ref, s0_ref), (W1, wr1_ref, s1_ref),
                 (W2, wr2_ref, s2_ref), (W710, wr710_ref, s710_ref),
                 (W912, wr912_ref, s912_ref))
        for (sc, wr, sr), g in zip(convs, _GEOM_CONV):
            _build_conv(sc, wr, sr, *g)
        ids = ((W5a, wr5a_ref, i5a_ref), (W5b, wr5b_ref, i5b_ref),
               (Wd0, wrd0_ref, id0_ref), (Wd1, wrd1_ref, id1_ref))
        for (sc, wr, ir), g in zip(ids, _GEOM_ID):
            _build_id(sc, wr, ir, *g)

    @pl.when(pid > 0)
    def _compute():
        def dot(a, b_ref):
            return jnp.dot(a, b_ref[...], preferred_element_type=f32)

        a0 = _silu_bf(dot(x0_ref[...].astype(_BF), W0) + b0_ref[...])
        a1 = _silu_bf(dot(a0, W1) + b1_ref[...])
        a2 = _silu_bf(dot(a1, W2) + b2_ref[...])
        a5 = _silu_bf(dot(a2, W5a) + dot(a1, W5b) + b5_ref[...])
        det0_ref[...] = dot(a5, Wd0) + bd0_ref[...]
        det1_ref[...] = dot(a2, Wd1) + bd1_ref[...]
        a710 = _silu_bf(dot(a5, W710) + b710_ref[...])
        seg = 1.0 / (1.0 + jnp.exp(-(dot(a710, W912) + b912_ref[...])))
        da_ref[...] = seg[:, 0:512]
        ll_ref[...] = seg[:, 512:1024]


def _const_spec(shape):
    return pl.BlockSpec(shape, lambda b: (0,) * len(shape))


def kernel(x, w0, b0, w1, b1, w2, b2, w5, b5, wd0, bd0, wd1, bd1,
           w710, b710, w912, b912):
    f32 = jnp.float32
    x = x.astype(f32)
    n = x.shape[0]
    bb = 256 if n % 256 == 0 else n
    nblk = n // bb

    w0r = w0.reshape(8, 9, 3).transpose(1, 0, 2)
    wreps = (_wrep_conv(w0r, 256), _wrep_conv(w1, 64), _wrep_conv(w2, 16),
             _wrep_conv(w710, 64), _wrep_conv(w912, 256),
             jnp.repeat(w5[:, :32].T, 64, axis=1),
             jnp.repeat(w5[:, 32:48].T, 64, axis=1),
             _wrep_det(wd0, 8), _wrep_det(wd1, 4))
    srcs = tuple(jnp.asarray(a) for a in
                 (_SRC0, _SRC1, _SRC2, _SRC710, _SRC912,
                  _ID5A, _ID5B, _IDD0, _IDD1))

    def brow(b, rep):
        return jnp.repeat(b.astype(f32), rep)[None, :]

    biases = (brow(b0, 256), brow(b1, 64), brow(b2, 16), brow(b5, 64),
              _det_bias_row(bd0, 8), _det_bias_row(bd1, 4),
              brow(b710, 64), brow(b912, 256))

    x0 = x.reshape(n, 3 * 1024)

    def xmap(b):
        return (jnp.maximum(b - 1, 0), 0)

    det0, det1, da, ll = pl.pallas_call(
        _mcnet_kernel,
        grid=(nblk + 1,),
        in_specs=([pl.BlockSpec((bb, 3072), xmap)]
                  + [_const_spec(a.shape) for a in wreps]
                  + [_const_spec(a.shape) for a in srcs]
                  + [_const_spec(a.shape) for a in biases]),
        out_specs=(
            pl.BlockSpec((bb, 2880), xmap),
            pl.BlockSpec((bb, 720), xmap),
            pl.BlockSpec((bb, 512), xmap),
            pl.BlockSpec((bb, 512), xmap),
        ),
        out_shape=(
            jax.ShapeDtypeStruct((n, 2880), f32),
            jax.ShapeDtypeStruct((n, 720), f32),
            jax.ShapeDtypeStruct((n, 512), f32),
            jax.ShapeDtypeStruct((n, 512), f32),
        ),
        scratch_shapes=[
            pltpu.VMEM((3072, 2048), _BF), pltpu.VMEM((2048, 1024), _BF),
            pltpu.VMEM((1024, 512), _BF), pltpu.VMEM((1024, 1024), _BF),
            pltpu.VMEM((1024, 1024), _BF), pltpu.VMEM((512, 1024), _BF),
            pltpu.VMEM((1024, 1024), _BF), pltpu.VMEM((1024, 2880), _BF),
            pltpu.VMEM((512, 720), _BF),
        ],
        compiler_params=pltpu.CompilerParams(
            dimension_semantics=("arbitrary",),
            vmem_limit_bytes=56 * 1024 * 1024),
    )(x0, *wreps, *srcs, *biases)

    det_out = [det0.reshape(n, 3, 8, 8, 15), det1.reshape(n, 3, 4, 4, 15)]
    return [det_out, da.reshape(n, 2, 16, 16), ll.reshape(n, 2, 16, 16)]


# speedup vs baseline: 1.0739x; 1.0618x over previous
import numpy as np

import jax
import jax.numpy as jnp
from jax.experimental import pallas as pl
from jax.experimental.pallas import tpu as pltpu

_BF = jnp.bfloat16


def _conv_srcmap(si, so, stride, cout):
    hwo = so * so
    m = np.full((9, hwo), -1, np.int32)
    for kh in range(3):
        for kw in range(3):
            t = kh * 3 + kw
            for r in range(so):
                ir = stride * r + kh - 1
                if not 0 <= ir < si:
                    continue
                for c in range(so):
                    ic = stride * c + kw - 1
                    if 0 <= ic < si:
                        m[t, r * so + c] = ir * si + ic
    return np.tile(m, (1, cout))


def _conv912_srcmap(cout):
    m = np.full((9, 256), -1, np.int32)
    for kh in range(3):
        for kw in range(3):
            t = kh * 3 + kw
            for r in range(16):
                ir = r + kh - 1
                if not 0 <= ir < 16:
                    continue
                for c in range(16):
                    ic = c + kw - 1
                    if 0 <= ic < 16:
                        m[t, r * 16 + c] = (ir // 2) * 8 + (ic // 2)
    return np.tile(m, (1, cout))


def _up4_idmap():
    m = np.zeros(64, np.int32)
    for r in range(8):
        for c in range(8):
            m[r * 8 + c] = (r // 2) * 4 + (c // 2)
    return np.tile(m, 16)[None, :]


_SRC0 = _conv_srcmap(32, 16, 2, 8)
_SRC1 = _conv_srcmap(16, 8, 2, 16)
_SRC2 = _conv_srcmap(8, 4, 2, 32)
_SRC710 = _conv_srcmap(8, 8, 1, 16)
_SRC912 = _conv912_srcmap(4)
_ID5A = _up4_idmap()
_ID5B = (np.arange(1024, dtype=np.int32) % 64)[None, :]
_IDD0 = ((np.arange(2880, dtype=np.int32) // 15) % 64)[None, :]
_IDD1 = ((np.arange(720, dtype=np.int32) // 15) % 16)[None, :]

_GEOM_CONV = ((1024, 3, 2048), (256, 8, 1024), (64, 16, 512),
              (64, 16, 1024), (64, 16, 1024))
_GEOM_ID = ((16, 32, 1024), (64, 16, 1024),
            (64, 16, 2880), (16, 32, 720))


def _wrep_conv(w, hwo):
    t, co, ci = w.shape
    r = jnp.repeat(w.transpose(0, 2, 1), hwo, axis=2)
    return r.reshape(t * ci, co * hwo)


def _wrep_det(wd, side):
    ci = wd.shape[1]
    a = wd.reshape(3, 15, ci).transpose(2, 0, 1)[:, :, None, :]
    return jnp.broadcast_to(a, (ci, 3, side * side, 15)).reshape(
        ci, 3 * side * side * 15)


def _det_bias_row(bd, side):
    hw = side * side
    return jnp.broadcast_to(bd.reshape(3, 1, 1, 15),
                            (3, side, side, 15)).reshape(1, 3 * hw * 15)


def _silu_bf(v):
    return (v * pl.reciprocal(1.0 + jnp.exp(-v), approx=True)).astype(_BF)


def _build_conv(scratch_ref, wrep_ref, src_ref, hwi, cin, cols):
    ch = min(64, hwi)

    def chunk(i, off, rows):
        iloc = off + jax.lax.broadcasted_iota(jnp.int32, (rows, cols), 0)
        acc = jnp.zeros((rows, cols), jnp.float32)
        for t in range(9):
            srow = src_ref[t:t + 1, :]
            r = t * cin + i
            wrow = wrep_ref[r:r + 1, :]
            acc = acc + jnp.where(iloc == srow,
                                  jnp.broadcast_to(wrow, (rows, cols)),
                                  jnp.zeros((rows, cols), jnp.float32))
        return acc.astype(_BF)

    for i in range(cin):
        if hwi > ch:
            def body(k, _, _i=i):
                off = pl.multiple_of(k * ch, ch)
                scratch_ref[pl.ds(_i * hwi + off, ch), :] = chunk(_i, off, ch)
                return 0
            jax.lax.fori_loop(0, hwi // ch, body, 0)
        else:
            scratch_ref[i * hwi:(i + 1) * hwi, :] = chunk(i, 0, hwi)


def _build_id(scratch_ref, wrep_ref, id_ref, hwi, cin, cols):
    iloc = jax.lax.broadcasted_iota(jnp.int32, (hwi, cols), 0)
    sel = iloc == id_ref[...]
    for i in range(cin):
        wrow = wrep_ref[i:i + 1, :]
        scratch_ref[i * hwi:(i + 1) * hwi, :] = jnp.where(
            sel, jnp.broadcast_to(wrow, (hwi, cols)),
            jnp.zeros((hwi, cols), jnp.float32)).astype(_BF)


def _mcnet_kernel(x0_ref,
                  wr0_ref, wr1_ref, wr2_ref, wr710_ref, wr912_ref,
                  wr5a_ref, wr5b_ref, wrd0_ref, wrd1_ref,
                  s0_ref, s1_ref, s2_ref, s710_ref, s912_ref,
                  i5a_ref, i5b_ref, id0_ref, id1_ref,
                  b0_ref, b1_ref, b2_ref, b5_ref,
                  bd0_ref, bd1_ref, b710_ref, b912_ref,
                  det0_ref, det1_ref, da_ref, ll_ref,
                  W0, W1, W2, W710, W912, W5a, W5b, Wd0, Wd1):
    f32 = jnp.float32
    pid = pl.program_id(0)

    @pl.when(pid == 0)
    def _build():
        convs = ((W0, wr0_ref, s0_ref), (W1, wr1_ref, s1_ref),
                 (W2, wr2_ref, s2_ref), (W710, wr710_ref, s710_ref),
                 (W912, wr912_ref, s912_ref))
        for (sc, wr, sr), g in zip(convs, _GEOM_CONV):
            _build_conv(sc, wr, sr, *g)
        ids = ((W5a, wr5a_ref, i5a_ref), (W5b, wr5b_ref, i5b_ref),
               (Wd0, wrd0_ref, id0_ref), (Wd1, wrd1_ref, id1_ref))
        for (sc, wr, ir), g in zip(ids, _GEOM_ID):
            _build_id(sc, wr, ir, *g)

    @pl.when(pid > 0)
    def _compute():
        def dot(a, b_ref):
            return jnp.dot(a, b_ref[...], preferred_element_type=f32)

        a0 = _silu_bf(dot(x0_ref[...].astype(_BF), W0) + b0_ref[...])
        a1 = _silu_bf(dot(a0, W1) + b1_ref[...])
        a2 = _silu_bf(dot(a1, W2) + b2_ref[...])
        a5 = _silu_bf(dot(a2, W5a) + dot(a1, W5b) + b5_ref[...])
        det0_ref[...] = dot(a5, Wd0) + bd0_ref[...]
        det1_ref[...] = dot(a2, Wd1) + bd1_ref[...]
        a710 = _silu_bf(dot(a5, W710) + b710_ref[...])
        seg = 1.0 / (1.0 + jnp.exp(-(dot(a710, W912) + b912_ref[...])))
        da_ref[...] = seg[:, 0:512]
        ll_ref[...] = seg[:, 512:1024]


def _const_spec(shape):
    return pl.BlockSpec(shape, lambda b: (0,) * len(shape))


def kernel(x, w0, b0, w1, b1, w2, b2, w5, b5, wd0, bd0, wd1, bd1,
           w710, b710, w912, b912):
    f32 = jnp.float32
    x = x.astype(f32)
    n = x.shape[0]
    bb = 256 if n % 256 == 0 else n
    nblk = n // bb

    w0r = w0.reshape(8, 9, 3).transpose(1, 0, 2)
    wreps = (_wrep_conv(w0r, 256), _wrep_conv(w1, 64), _wrep_conv(w2, 16),
             _wrep_conv(w710, 64), _wrep_conv(w912, 256),
             jnp.repeat(w5[:, :32].T, 64, axis=1),
             jnp.repeat(w5[:, 32:48].T, 64, axis=1),
             _wrep_det(wd0, 8), _wrep_det(wd1, 4))
    srcs = tuple(jnp.asarray(a) for a in
                 (_SRC0, _SRC1, _SRC2, _SRC710, _SRC912,
                  _ID5A, _ID5B, _IDD0, _IDD1))

    def brow(b, rep):
        return jnp.repeat(b.astype(f32), rep)[None, :]

    biases = (brow(b0, 256), brow(b1, 64), brow(b2, 16), brow(b5, 64),
              _det_bias_row(bd0, 8), _det_bias_row(bd1, 4),
              brow(b710, 64), brow(b912, 256))

    x0 = x.reshape(n, 3 * 1024)

    def xmap(b):
        return (jnp.maximum(b - 1, 0), 0)

    det0, det1, da, ll = pl.pallas_call(
        _mcnet_kernel,
        grid=(nblk + 1,),
        in_specs=([pl.BlockSpec((bb, 3072), xmap)]
                  + [_const_spec(a.shape) for a in wreps]
                  + [_const_spec(a.shape) for a in srcs]
                  + [_const_spec(a.shape) for a in biases]),
        out_specs=(
            pl.BlockSpec((bb, 2880), xmap),
            pl.BlockSpec((bb, 720), xmap),
            pl.BlockSpec((bb, 512), xmap),
            pl.BlockSpec((bb, 512), xmap),
        ),
        out_shape=(
            jax.ShapeDtypeStruct((n, 2880), f32),
            jax.ShapeDtypeStruct((n, 720), f32),
            jax.ShapeDtypeStruct((n, 512), f32),
            jax.ShapeDtypeStruct((n, 512), f32),
        ),
        scratch_shapes=[
            pltpu.VMEM((3072, 2048), _BF), pltpu.VMEM((2048, 1024), _BF),
            pltpu.VMEM((1024, 512), _BF), pltpu.VMEM((1024, 1024), _BF),
            pltpu.VMEM((1024, 1024), _BF), pltpu.VMEM((512, 1024), _BF),
            pltpu.VMEM((1024, 1024), _BF), pltpu.VMEM((1024, 2880), _BF),
            pltpu.VMEM((512, 720), _BF),
        ],
        compiler_params=pltpu.CompilerParams(
            dimension_semantics=("arbitrary",),
            vmem_limit_bytes=56 * 1024 * 1024),
    )(x0, *wreps, *srcs, *biases)

    det_out = [det0.reshape(n, 3, 8, 8, 15), det1.reshape(n, 3, 4, 4, 15)]
    return [det_out, da.reshape(n, 2, 16, 16), ll.reshape(n, 2, 16, 16)]
